# Initial kernel scaffold; baseline (speedup 1.0000x reference)
import jax
import jax.numpy as jnp
from jax.experimental import pallas as pl


def kernel(x, edge_attr, params, edge_index, batch, num_graphs):
    raise NotImplementedError("write your pallas kernel here")


# speedup vs baseline: 2.2629x; 2.2629x over previous
import functools

import jax
import jax.numpy as jnp
from jax import lax
from jax.experimental import pallas as pl
from jax.experimental.pallas import tpu as pltpu
from jax.experimental.pallas import tpu_sc as plsc

F32 = jnp.float32

N = 10000
E = 320000
G = 64

BN = 1000
BE = 4000

NC, NS = 2, 16
W = NC * NS
EW = E // W
C = 400
NCH = EW // C
RT = N // NS


def _mesh():
    return plsc.VectorSubcoreMesh(
        core_axis_name="c", subcore_axis_name="s", num_cores=NC, num_subcores=NS)


CH = 128
NCHT = E // CH


def _sc_gather_l1(row, col, r128, tc):
    def body(row_h, col_h, a_h, b_h, gs_o, gc_o, idr, idc, ba, bb, sem):
        wid = lax.axis_index("s") * NC + lax.axis_index("c")
        nch = NCHT // W + jnp.where(wid < NCHT % W, 1, 0)

        def step(k, carry):
            base = (k * W + wid) * CH
            pltpu.sync_copy(row_h.at[pl.ds(base, CH)], idr)
            pltpu.sync_copy(col_h.at[pl.ds(base, CH)], idc)
            d1 = pltpu.async_copy(a_h.at[idr], ba, sem)
            d2 = pltpu.async_copy(b_h.at[idc], bb, sem)
            d1.wait(); d2.wait()
            pltpu.sync_copy(ba, gs_o.at[pl.ds(base, CH)])
            pltpu.sync_copy(bb, gc_o.at[pl.ds(base, CH)])
            return carry

        lax.fori_loop(0, nch, step, 0)

    f = pl.kernel(
        body,
        out_type=(jax.ShapeDtypeStruct((E, 128), F32),
                  jax.ShapeDtypeStruct((E, 256), F32)),
        mesh=_mesh(),
        scratch_types=[pltpu.VMEM((CH,), jnp.int32), pltpu.VMEM((CH,), jnp.int32),
                       pltpu.VMEM((CH, 128), F32), pltpu.VMEM((CH, 256), F32),
                       pltpu.SemaphoreType.DMA])
    return f(row, col, r128, tc)


def _sc_gather_l2(row, col, q2s, q2d):
    def body(row_h, col_h, a_h, b_h, gs_o, gd_o, idr, idc, ba, bb, sem):
        wid = lax.axis_index("s") * NC + lax.axis_index("c")
        nch = NCHT // W + jnp.where(wid < NCHT % W, 1, 0)

        def step(k, carry):
            base = (k * W + wid) * CH
            pltpu.sync_copy(row_h.at[pl.ds(base, CH)], idr)
            pltpu.sync_copy(col_h.at[pl.ds(base, CH)], idc)
            d1 = pltpu.async_copy(a_h.at[idr], ba, sem)
            d2 = pltpu.async_copy(b_h.at[idc], bb, sem)
            d1.wait(); d2.wait()
            pltpu.sync_copy(ba, gs_o.at[pl.ds(base, CH)])
            pltpu.sync_copy(bb, gd_o.at[pl.ds(base, CH)])
            return carry

        lax.fori_loop(0, nch, step, 0)

    f = pl.kernel(
        body,
        out_type=(jax.ShapeDtypeStruct((E, 128), F32),
                  jax.ShapeDtypeStruct((E, 128), F32)),
        mesh=_mesh(),
        scratch_types=[pltpu.VMEM((CH,), jnp.int32), pltpu.VMEM((CH,), jnp.int32),
                       pltpu.VMEM((CH, 128), F32), pltpu.VMEM((CH, 128), F32),
                       pltpu.SemaphoreType.DMA])
    return f(row, col, q2s, q2d)


def _remap_idx(idx, idx2, lo, nh):
    def remap(i, carry):
        v = idx[pl.ds(i * 16, 16)] - lo
        oob = (v < 0) | (v >= nh)
        trash = nh + (v & 7)
        idx2[pl.ds(i * 16, 16)] = jnp.where(oob, trash, v)
        return carry

    lax.fori_loop(0, CH // 16, remap, 0)


def _sc_scatter(row, tt, z128, o128):
    NH = N // NC
    RB = 312

    def body(row_h, tt_h, z128_h, o128_h, a_o, c_o, idx, idx2, buf, accS, sem):
        cid = lax.axis_index("c")
        sid = lax.axis_index("s")
        lo = cid * NH
        rbase = sid * RB
        nch = NCHT // NS + jnp.where(sid < NCHT % NS, 1, 0)

        def zero_window():
            pltpu.sync_copy(z128_h, buf)
            for (o, sz) in ((0, 128), (128, 128), (256, 72)):
                pltpu.sync_copy(buf.at[pl.ds(0, sz)], accS.at[pl.ds(rbase + o, sz)])

        def stage_out(dst):
            for (o, sz) in ((0, 128), (128, 128), (256, 64)):
                pltpu.sync_copy(accS.at[pl.ds(rbase + o, sz)], buf.at[pl.ds(0, sz)])
                pltpu.sync_copy(buf.at[pl.ds(0, sz)], dst.at[pl.ds(lo + rbase + o, sz)])

        zero_window()
        plsc.subcore_barrier()

        def step(k, carry):
            base = (k * NS + sid) * CH
            pltpu.sync_copy(row_h.at[pl.ds(base, CH)], idx)
            pltpu.sync_copy(tt_h.at[pl.ds(base, CH)], buf)
            _remap_idx(idx, idx2, lo, NH)
            pltpu.sync_copy(buf, accS.at[idx2], add=True)
            return carry

        lax.fori_loop(0, nch, step, 0)
        plsc.subcore_barrier()
        stage_out(a_o)
        plsc.subcore_barrier()

        zero_window()
        plsc.subcore_barrier()

        def step2(k, carry):
            base = (k * NS + sid) * CH
            pltpu.sync_copy(row_h.at[pl.ds(base, CH)], idx)
            pltpu.sync_copy(o128_h, buf)
            _remap_idx(idx, idx2, lo, NH)
            pltpu.sync_copy(buf, accS.at[idx2], add=True)
            return carry

        lax.fori_loop(0, nch, step2, 0)
        plsc.subcore_barrier()
        stage_out(c_o)

    f = pl.kernel(
        body,
        out_type=(jax.ShapeDtypeStruct((N, 128), F32),
                  jax.ShapeDtypeStruct((N, 128), F32)),
        mesh=_mesh(),
        scratch_types=[pltpu.VMEM((CH,), jnp.int32),
                       pltpu.VMEM((CH,), jnp.int32),
                       pltpu.VMEM((CH, 128), F32),
                       pltpu.VMEM_SHARED((N // NC + 8, 128), F32),
                       pltpu.SemaphoreType.DMA])
    return f(row, tt, z128, o128)


def _dot(a, b):
    return jnp.dot(a, b, preferred_element_type=F32,
                   precision=lax.Precision.HIGHEST)


def _k_a(x, batchf, w1s, w1d, wn1ax, we1b, wn1ae, be1b, bn1a):
    def body(x_r, bf_r, w1s_r, w1d_r, wnx_r, we1b_r, wne_r, be1b_r, bn1a_r,
             r128_r, tc_r, m1_r, c1_r):
        xb = x_r[...]
        r128_r[...] = jnp.concatenate(
            [_dot(xb, w1s_r[...]), bf_r[...], jnp.zeros((BN, 63), F32)], axis=1)
        tc_r[...] = jnp.concatenate(
            [_dot(xb, w1d_r[...]), _dot(xb, wnx_r[...]),
             jnp.zeros((BN, 64), F32)], axis=1)
        m1_r[...] = _dot(we1b_r[...], wne_r[...])
        c1_r[...] = _dot(be1b_r[...], wne_r[...]) + bn1a_r[...]

    cst = lambda i: (0, 0)
    blk = lambda i: (i, 0)
    return pl.pallas_call(
        body,
        grid=(N // BN,),
        in_specs=[pl.BlockSpec((BN, 128), blk), pl.BlockSpec((BN, 1), blk),
                  pl.BlockSpec((128, 64), cst), pl.BlockSpec((128, 64), cst),
                  pl.BlockSpec((128, 128), cst), pl.BlockSpec((64, 128), cst),
                  pl.BlockSpec((128, 128), cst), pl.BlockSpec((1, 128), cst),
                  pl.BlockSpec((1, 128), cst)],
        out_specs=[pl.BlockSpec((BN, 128), blk), pl.BlockSpec((BN, 256), blk),
                   pl.BlockSpec((64, 128), cst), pl.BlockSpec((1, 128), cst)],
        out_shape=[jax.ShapeDtypeStruct((N, 128), F32),
                   jax.ShapeDtypeStruct((N, 256), F32),
                   jax.ShapeDtypeStruct((64, 128), F32),
                   jax.ShapeDtypeStruct((1, 128), F32)],
    )(x, batchf, w1s, w1d, wn1ax, we1b, wn1ae, be1b, bn1a)


def _k_b(gs, gc, ea, we1ae, be1a, m1, c1):
    def body(gs_r, gc_r, ea_r, we_r, be1a_r, m1_r, c1_r,
             t1_r, tt_r, brof_r, g1_r, a_r):
        i = pl.program_id(0)
        gs = gs_r[...]
        gc = gc_r[...]
        pre = (gs[:, 0:64] + gc[:, 0:64] + _dot(ea_r[...], we_r[...])
               + be1a_r[...])
        t1 = jnp.maximum(pre, 0.0)
        tt = jnp.maximum(gc[:, 64:192] + _dot(t1, m1_r[...]) + c1_r[...], 0.0)
        t1_r[...] = t1
        tt_r[...] = tt
        brof = gs[:, 64:65]
        brof_r[...] = brof
        iota = lax.broadcasted_iota(jnp.int32, (1, G), 1).astype(F32)
        oh = (brof == iota).astype(F32)
        cat = jnp.concatenate([t1, jnp.ones((BE, 64), F32)], axis=1)
        g1u = lax.dot_general(oh, cat, (((0,), (0,)), ((), ())),
                              preferred_element_type=F32,
                              precision=lax.Precision.HIGHEST)
        au = lax.dot_general(t1, t1, (((0,), (0,)), ((), ())),
                             preferred_element_type=F32,
                             precision=lax.Precision.HIGHEST)

        @pl.when(i == 0)
        def _():
            g1_r[...] = jnp.zeros((G, 128), F32)
            a_r[...] = jnp.zeros((64, 64), F32)

        g1_r[...] += g1u
        a_r[...] += au

    cst = lambda i: (0, 0)
    blk = lambda i: (i, 0)
    return pl.pallas_call(
        body,
        grid=(E // BE,),
        in_specs=[pl.BlockSpec((BE, 128), blk), pl.BlockSpec((BE, 256), blk),
                  pl.BlockSpec((BE, 16), blk),
                  pl.BlockSpec((16, 64), cst), pl.BlockSpec((1, 64), cst),
                  pl.BlockSpec((64, 128), cst), pl.BlockSpec((1, 128), cst)],
        out_specs=[pl.BlockSpec((BE, 64), blk), pl.BlockSpec((BE, 128), blk),
                   pl.BlockSpec((BE, 1), blk), pl.BlockSpec((G, 128), cst),
                   pl.BlockSpec((64, 64), cst)],
        out_shape=[jax.ShapeDtypeStruct((E, 64), F32),
                   jax.ShapeDtypeStruct((E, 128), F32),
                   jax.ShapeDtypeStruct((E, 1), F32),
                   jax.ShapeDtypeStruct((G, 128), F32),
                   jax.ShapeDtypeStruct((64, 64), F32)],
    )(gs, gc, ea, we1ae, be1a, m1, c1)


def _k_d(a0, c0, x, wn1b, bn1b, wn1cx, wn1ca, bn1c, wn1d, bn1d):
    def body(a0_r, c0_r, x_r, wn1b_r, bn1b_r, wcx_r, wca_r,
             bn1c_r, wn1d_r, bn1d_r, x2_r, xs_r):
        i = pl.program_id(0)
        cnt = c0_r[...][:, 0:1]
        cc = jnp.maximum(cnt, 1.0)
        maskn = (cnt > 0).astype(F32)
        mt = a0_r[...] / cc
        agg = _dot(mt, wn1b_r[...]) + maskn * bn1b_r[...]
        h1 = jnp.maximum(
            _dot(x_r[...], wcx_r[...]) + _dot(agg, wca_r[...]) + bn1c_r[...], 0.0)
        x2 = _dot(h1, wn1d_r[...]) + bn1d_r[...]
        x2_r[...] = x2

        @pl.when(i == 0)
        def _():
            xs_r[...] = jnp.zeros((2, 256), F32)

        xs_r[...] += jnp.concatenate(
            [jnp.sum(x2, axis=0, keepdims=True),
             jnp.sum(x2 * x2, axis=0, keepdims=True)], axis=0)

    cst = lambda i: (0, 0)
    blk = lambda i: (i, 0)
    return pl.pallas_call(
        body,
        grid=(N // BN,),
        in_specs=[pl.BlockSpec((BN, 128), blk), pl.BlockSpec((BN, 128), blk),
                  pl.BlockSpec((BN, 128), blk),
                  pl.BlockSpec((128, 128), cst), pl.BlockSpec((1, 128), cst),
                  pl.BlockSpec((128, 128), cst), pl.BlockSpec((128, 128), cst),
                  pl.BlockSpec((1, 128), cst), pl.BlockSpec((128, 256), cst),
                  pl.BlockSpec((1, 256), cst)],
        out_specs=[pl.BlockSpec((BN, 256), blk), pl.BlockSpec((2, 256), cst)],
        out_shape=[jax.ShapeDtypeStruct((N, 256), F32),
                   jax.ShapeDtypeStruct((2, 256), F32)],
    )(a0, c0, x, wn1b, bn1b, wn1cx, wn1ca, bn1c, wn1d, bn1d)


def _k_d2(x2, xs, we2as, we2ad, gn, bnn):
    def body(x2_r, xs_r, ws_r, wd_r, gn_r, bnn_r, qs_r, qd_r, csd_r):
        xs = xs_r[...]
        mx = xs[0:1] * (1.0 / N)
        vx = xs[1:2] * (1.0 / N) - mx * mx
        sx = gn_r[...] * lax.rsqrt(vx + 1e-5)
        tn = bnn_r[...] - mx * sx
        x2s = x2_r[...] * sx
        qs_r[...] = _dot(x2s, ws_r[...])
        qd_r[...] = _dot(x2s, wd_r[...])
        csd_r[...] = _dot(tn, ws_r[...]) + _dot(tn, wd_r[...])

    cst = lambda i: (0, 0)
    blk = lambda i: (i, 0)
    return pl.pallas_call(
        body,
        grid=(N // BN,),
        in_specs=[pl.BlockSpec((BN, 256), blk), pl.BlockSpec((2, 256), cst),
                  pl.BlockSpec((256, 128), cst), pl.BlockSpec((256, 128), cst),
                  pl.BlockSpec((1, 256), cst), pl.BlockSpec((1, 256), cst)],
        out_specs=[pl.BlockSpec((BN, 128), blk), pl.BlockSpec((BN, 128), blk),
                   pl.BlockSpec((1, 128), cst)],
        out_shape=[jax.ShapeDtypeStruct((N, 128), F32),
                   jax.ShapeDtypeStruct((N, 128), F32),
                   jax.ShapeDtypeStruct((1, 128), F32)],
    )(x2, xs, we2as, we2ad, gn, bnn)


def _k_w(g1acc, aacc, we1b, be1b, ge, bee, we2ae, be2a):
    def body(g1_r, a_r, we1b_r, be1b_r, ge_r, bee_r, wee_r, be2a_r, m2_r, c2e_r):
        we1b = we1b_r[...]
        be1b = be1b_r[...]
        s1 = jnp.sum(g1_r[...][:, 0:64], axis=0, keepdims=True)
        s1w = _dot(s1, we1b)
        me = (s1w + E * be1b) * (1.0 / E)
        d1 = _dot(a_r[...], we1b)
        diag = jnp.sum(we1b * d1, axis=0, keepdims=True)
        sumsq = diag + 2.0 * be1b * s1w + E * be1b * be1b
        ve = sumsq * (1.0 / E) - me * me
        se = ge_r[...] * lax.rsqrt(ve + 1e-5)
        te = bee_r[...] - me * se
        m2_r[...] = _dot(we1b * se, wee_r[...])
        c2e_r[...] = _dot(be1b * se + te, wee_r[...]) + be2a_r[...]

    return pl.pallas_call(
        body,
        out_shape=[jax.ShapeDtypeStruct((64, 128), F32),
                   jax.ShapeDtypeStruct((1, 128), F32)],
    )(g1acc, aacc, we1b, be1b, ge, bee, we2ae, be2a)


def _k_e(g2s, g2d, t1, brof, m2, c2e, c2sd):
    def body(gs_r, gd_r, t1_r, brof_r, m2_r, c2e_r, csd_r, g2_r):
        i = pl.program_id(0)
        pre = (gs_r[...] + gd_r[...] + _dot(t1_r[...], m2_r[...])
               + c2e_r[...] + csd_r[...])
        t2 = jnp.maximum(pre, 0.0)
        iota = lax.broadcasted_iota(jnp.int32, (1, G), 1).astype(F32)
        oh = (brof_r[...] == iota).astype(F32)
        g2u = lax.dot_general(oh, t2, (((0,), (0,)), ((), ())),
                              preferred_element_type=F32,
                              precision=lax.Precision.HIGHEST)

        @pl.when(i == 0)
        def _():
            g2_r[...] = jnp.zeros((G, 128), F32)

        g2_r[...] += g2u

    cst = lambda i: (0, 0)
    blk = lambda i: (i, 0)
    return pl.pallas_call(
        body,
        grid=(E // BE,),
        in_specs=[pl.BlockSpec((BE, 128), blk), pl.BlockSpec((BE, 128), blk),
                  pl.BlockSpec((BE, 64), blk), pl.BlockSpec((BE, 1), blk),
                  pl.BlockSpec((64, 128), cst), pl.BlockSpec((1, 128), cst),
                  pl.BlockSpec((1, 128), cst)],
        out_specs=[pl.BlockSpec((G, 128), cst)],
        out_shape=[jax.ShapeDtypeStruct((G, 128), F32)],
    )(g2s, g2d, t1, brof, m2, c2e, c2sd)


def _k_f(g1acc, g2acc, we1b, be1b, we2b, be2b, wg1ag, bg1a, wg1b, bg1b,
         gu, bu, wg2au, wg2ag, bg2a, wg2b, bg2b, wf1, bf1, wf2, bf2):
    def body(g1_r, g2_r, we1b_r, be1b_r, we2b_r, be2b_r, wg1ag_r, bg1a_r,
             wg1b_r, bg1b_r, gu_r, bu_r, wg2au_r, wg2ag_r, bg2a_r, wg2b_r,
             bg2b_r, wf1_r, bf1_r, wf2_r, bf2_r, out_r):
        g1acc = g1_r[...]
        gcnt = g1acc[:, 64:65]
        gc = jnp.maximum(gcnt, 1.0)
        mask = (gcnt > 0).astype(F32)
        g1 = _dot(g1acc[:, 0:64] / gc, we1b_r[...]) + mask * be1b_r[...]
        u1h = jnp.maximum(_dot(g1, wg1ag_r[...]) + bg1a_r[...], 0.0)
        u1 = _dot(u1h, wg1b_r[...]) + bg1b_r[...]
        m = jnp.mean(u1, axis=0, keepdims=True)
        v = jnp.mean((u1 - m) * (u1 - m), axis=0, keepdims=True)
        u1 = (u1 - m) * lax.rsqrt(v + 1e-5) * gu_r[...] + bu_r[...]
        g2 = _dot(g2_r[...] / gc, we2b_r[...]) + mask * be2b_r[...]
        u2h = jnp.maximum(
            _dot(u1, wg2au_r[...]) + _dot(g2, wg2ag_r[...]) + bg2a_r[...], 0.0)
        u2 = _dot(u2h, wg2b_r[...]) + bg2b_r[...]
        f1 = jnp.maximum(_dot(u2, wf1_r[...]) + bf1_r[...], 0.0)
        out_r[...] = _dot(f1, wf2_r[...]) + bf2_r[...]

    return pl.pallas_call(
        body,
        out_shape=jax.ShapeDtypeStruct((G, 1), F32),
    )(g1acc, g2acc, we1b, be1b, we2b, be2b, wg1ag, bg1a, wg1b, bg1b,
      gu, bu, wg2au, wg2ag, bg2a, wg2b, bg2b, wf1, bf1, wf2, bf2)


def kernel(x, edge_attr, params, edge_index, batch, num_graphs):
    p = params
    assert x.shape == (N, 128) and edge_index.shape == (2, E)

    row = edge_index[0]
    col = edge_index[1]
    batchf = batch.astype(F32).reshape(N, 1)

    r1 = lambda b: b.reshape(1, -1)

    r128, tc, m1, c1 = _k_a(
        x, batchf,
        p["We1a"][0:128], p["We1a"][128:256], p["Wn1a"][0:128],
        p["We1b"], p["Wn1a"][128:256], r1(p["be1b"]), r1(p["bn1a"]))

    gs, gc = _sc_gather_l1(row, col, r128, tc)

    t1, tt, brof, g1acc, aacc = _k_b(
        gs, gc, edge_attr,
        p["We1a"][256:272], r1(p["be1a"]), m1, c1)

    z128 = jnp.zeros((CH, 128), F32)
    o128 = jnp.ones((CH, 128), F32)
    a0, cn0 = _sc_scatter(row, tt, z128, o128)

    x2, xs = _k_d(
        a0, cn0, x,
        p["Wn1b"], r1(p["bn1b"]), p["Wn1c"][0:128], p["Wn1c"][128:256],
        r1(p["bn1c"]), p["Wn1d"], r1(p["bn1d"]))

    q2s, q2d, c2sd = _k_d2(
        x2, xs, p["We2a"][0:256], p["We2a"][256:512],
        r1(p["g_node"]), r1(p["b_node"]))

    m2, c2e = _k_w(
        g1acc, aacc, p["We1b"], r1(p["be1b"]),
        r1(p["g_edge"]), r1(p["b_edge"]), p["We2a"][512:640], r1(p["be2a"]))

    g2s, g2d = _sc_gather_l2(row, col, q2s, q2d)

    (g2acc,) = _k_e(g2s, g2d, t1, brof, m2, c2e, c2sd)

    out = _k_f(
        g1acc, g2acc, p["We1b"], r1(p["be1b"]), p["We2b"], r1(p["be2b"]),
        p["Wg1a"][64:192], r1(p["bg1a"]), p["Wg1b"], r1(p["bg1b"]),
        r1(p["g_u"]), r1(p["b_u"]),
        p["Wg2a"][0:64], p["Wg2a"][64:192], r1(p["bg2a"]), p["Wg2b"],
        r1(p["bg2b"]), p["Wf1"], r1(p["bf1"]), p["Wf2"], r1(p["bf2"]))
    return out


# speedup vs baseline: 3.1667x; 1.3994x over previous
import functools

import jax
import jax.numpy as jnp
from jax import lax
from jax.experimental import pallas as pl
from jax.experimental.pallas import tpu as pltpu
from jax.experimental.pallas import tpu_sc as plsc

F32 = jnp.float32

N = 10000
E = 320000
G = 64

BN = 1000
BE = 4000

NC, NS = 2, 16
W = NC * NS
EW = E // W
C = 400
NCH = EW // C
RT = N // NS


def _mesh():
    return plsc.VectorSubcoreMesh(
        core_axis_name="c", subcore_axis_name="s", num_cores=NC, num_subcores=NS)


CH = 128
NCHT = E // CH


def _sc_gather2(row, col, ta, tb, wa, wb):
    def body(row_h, col_h, a_h, b_h, oa_o, ob_o,
             idr0, idr1, idc0, idc1, ba0, ba1, bb0, bb1, sg0, sg1, sw0, sw1):
        idr, idc = (idr0, idr1), (idc0, idc1)
        ba, bb = (ba0, ba1), (bb0, bb1)
        sg, sw = (sg0, sg1), (sw0, sw1)
        wid = lax.axis_index("s") * NC + lax.axis_index("c")
        nch = NCHT // W + jnp.where(wid < NCHT % W, 1, 0)

        def base_of(k):
            return (k * W + wid) * CH

        def load_start(k, b):
            base = base_of(k)
            pltpu.sync_copy(row_h.at[pl.ds(base, CH)], idr[b])
            pltpu.sync_copy(col_h.at[pl.ds(base, CH)], idc[b])
            pltpu.async_copy(a_h.at[idr[b]], ba[b], sg[b])
            pltpu.async_copy(b_h.at[idc[b]], bb[b], sg[b])

        def gather_wait(b):
            pltpu.make_async_copy(a_h.at[idr[b]], ba[b], sg[b]).wait()
            pltpu.make_async_copy(b_h.at[idc[b]], bb[b], sg[b]).wait()

        def write_start(k, b):
            base = base_of(k)
            pltpu.async_copy(ba[b], oa_o.at[pl.ds(base, CH)], sw[b])
            pltpu.async_copy(bb[b], ob_o.at[pl.ds(base, CH)], sw[b])

        def write_wait(k, b):
            base = base_of(k)
            pltpu.make_async_copy(ba[b], oa_o.at[pl.ds(base, CH)], sw[b]).wait()
            pltpu.make_async_copy(bb[b], ob_o.at[pl.ds(base, CH)], sw[b]).wait()

        for b in range(2):
            @pl.when(b < nch)
            def _(b=b):
                load_start(b, b)

        def outer(ss, carry):
            for b in range(2):
                k = 2 * ss + b

                @pl.when(k < nch)
                def _(b=b, k=k):
                    gather_wait(b)
                    write_start(k, b)
                    write_wait(k, b)

                    @pl.when(k + 2 < nch)
                    def _(b=b, k=k):
                        load_start(k + 2, b)
            return carry

        lax.fori_loop(0, (nch + 1) // 2, outer, 0)

    f = pl.kernel(
        body,
        out_type=(jax.ShapeDtypeStruct((E, wa), F32),
                  jax.ShapeDtypeStruct((E, wb), F32)),
        mesh=_mesh(),
        scratch_types=[pltpu.VMEM((CH,), jnp.int32), pltpu.VMEM((CH,), jnp.int32),
                       pltpu.VMEM((CH,), jnp.int32), pltpu.VMEM((CH,), jnp.int32),
                       pltpu.VMEM((CH, wa), F32), pltpu.VMEM((CH, wa), F32),
                       pltpu.VMEM((CH, wb), F32), pltpu.VMEM((CH, wb), F32),
                       pltpu.SemaphoreType.DMA, pltpu.SemaphoreType.DMA,
                       pltpu.SemaphoreType.DMA, pltpu.SemaphoreType.DMA])
    return f(row, col, ta, tb)


def _sc_gather_l1(row, col, r128, tc):
    return _sc_gather2(row, col, r128, tc, 128, 256)


def _sc_gather_l2(row, col, q2s, q2d):
    return _sc_gather2(row, col, q2s, q2d, 128, 128)


def _remap_idx(idx, idx2, lo, nh):
    def remap(i, carry):
        v = idx[pl.ds(i * 16, 16)] - lo
        oob = (v < 0) | (v >= nh)
        trash = nh + (v & 7)
        idx2[pl.ds(i * 16, 16)] = jnp.where(oob, trash, v)
        return carry

    lax.fori_loop(0, CH // 16, remap, 0)


def _sc_scatter(row, tt, z128, o128):
    NH = N // NC
    RB = 312

    def body(row_h, tt_h, z128_h, o128_h, a_o, c_o,
             idx0, idx1, idp0, idp1, buf0, buf1, obuf, accS, sl0, sl1):
        idx, idp = (idx0, idx1), (idp0, idp1)
        buf, sl = (buf0, buf1), (sl0, sl1)
        cid = lax.axis_index("c")
        sid = lax.axis_index("s")
        lo = cid * NH
        rbase = sid * RB
        nch = NCHT // NS + jnp.where(sid < NCHT % NS, 1, 0)

        def base_of(k):
            return (k * NS + sid) * CH

        def zero_window():
            pltpu.sync_copy(z128_h, buf0)
            for (o, sz) in ((0, 128), (128, 128), (256, 72)):
                pltpu.sync_copy(buf0.at[pl.ds(0, sz)], accS.at[pl.ds(rbase + o, sz)])

        def stage_out(dst):
            for (o, sz) in ((0, 128), (128, 128), (256, 64)):
                pltpu.sync_copy(accS.at[pl.ds(rbase + o, sz)], buf0.at[pl.ds(0, sz)])
                pltpu.sync_copy(buf0.at[pl.ds(0, sz)], dst.at[pl.ds(lo + rbase + o, sz)])

        pltpu.sync_copy(o128_h, obuf)
        zero_window()
        plsc.subcore_barrier()

        def load1(k, b):
            base = base_of(k)
            pltpu.async_copy(row_h.at[pl.ds(base, CH)], idx[b], sl[b])
            pltpu.async_copy(tt_h.at[pl.ds(base, CH)], buf[b], sl[b])

        def wait1(k, b):
            base = base_of(k)
            pltpu.make_async_copy(row_h.at[pl.ds(base, CH)], idx[b], sl[b]).wait()
            pltpu.make_async_copy(tt_h.at[pl.ds(base, CH)], buf[b], sl[b]).wait()

        for b in range(2):
            @pl.when(b < nch)
            def _(b=b):
                load1(b, b)

        def outer1(ss, carry):
            for b in range(2):
                k = 2 * ss + b

                @pl.when(k < nch)
                def _(b=b, k=k):
                    wait1(k, b)
                    _remap_idx(idx[b], idp[b], lo, NH)
                    pltpu.sync_copy(buf[b], accS.at[idp[b]], add=True)

                    @pl.when(k + 2 < nch)
                    def _(b=b, k=k):
                        load1(k + 2, b)
            return carry

        lax.fori_loop(0, (nch + 1) // 2, outer1, 0)
        plsc.subcore_barrier()
        stage_out(a_o)
        plsc.subcore_barrier()

        zero_window()
        plsc.subcore_barrier()

        def load2(k, b):
            pltpu.async_copy(row_h.at[pl.ds(base_of(k), CH)], idx[b], sl[b])

        def wait2(k, b):
            pltpu.make_async_copy(
                row_h.at[pl.ds(base_of(k), CH)], idx[b], sl[b]).wait()

        for b in range(2):
            @pl.when(b < nch)
            def _(b=b):
                load2(b, b)

        def outer2(ss, carry):
            for b in range(2):
                k = 2 * ss + b

                @pl.when(k < nch)
                def _(b=b, k=k):
                    wait2(k, b)
                    _remap_idx(idx[b], idp[b], lo, NH)
                    pltpu.sync_copy(obuf, accS.at[idp[b]], add=True)

                    @pl.when(k + 2 < nch)
                    def _(b=b, k=k):
                        load2(k + 2, b)
            return carry

        lax.fori_loop(0, (nch + 1) // 2, outer2, 0)
        plsc.subcore_barrier()
        stage_out(c_o)

    f = pl.kernel(
        body,
        out_type=(jax.ShapeDtypeStruct((N, 128), F32),
                  jax.ShapeDtypeStruct((N, 128), F32)),
        mesh=_mesh(),
        scratch_types=[pltpu.VMEM((CH,), jnp.int32), pltpu.VMEM((CH,), jnp.int32),
                       pltpu.VMEM((CH,), jnp.int32), pltpu.VMEM((CH,), jnp.int32),
                       pltpu.VMEM((CH, 128), F32), pltpu.VMEM((CH, 128), F32),
                       pltpu.VMEM((CH, 128), F32),
                       pltpu.VMEM_SHARED((N // NC + 8, 128), F32),
                       pltpu.SemaphoreType.DMA, pltpu.SemaphoreType.DMA])
    return f(row, tt, z128, o128)


def _dot(a, b):
    return jnp.dot(a, b, preferred_element_type=F32,
                   precision=lax.Precision.HIGHEST)


def _k_a(x, batchf, w1s, w1d, wn1ax, we1b, wn1ae, be1b, bn1a):
    def body(x_r, bf_r, w1s_r, w1d_r, wnx_r, we1b_r, wne_r, be1b_r, bn1a_r,
             r128_r, tc_r, m1_r, c1_r):
        xb = x_r[...]
        r128_r[...] = jnp.concatenate(
            [_dot(xb, w1s_r[...]), bf_r[...], jnp.zeros((BN, 63), F32)], axis=1)
        tc_r[...] = jnp.concatenate(
            [_dot(xb, w1d_r[...]), _dot(xb, wnx_r[...]),
             jnp.zeros((BN, 64), F32)], axis=1)
        m1_r[...] = _dot(we1b_r[...], wne_r[...])
        c1_r[...] = _dot(be1b_r[...], wne_r[...]) + bn1a_r[...]

    cst = lambda i: (0, 0)
    blk = lambda i: (i, 0)
    return pl.pallas_call(
        body,
        grid=(N // BN,),
        in_specs=[pl.BlockSpec((BN, 128), blk), pl.BlockSpec((BN, 1), blk),
                  pl.BlockSpec((128, 64), cst), pl.BlockSpec((128, 64), cst),
                  pl.BlockSpec((128, 128), cst), pl.BlockSpec((64, 128), cst),
                  pl.BlockSpec((128, 128), cst), pl.BlockSpec((1, 128), cst),
                  pl.BlockSpec((1, 128), cst)],
        out_specs=[pl.BlockSpec((BN, 128), blk), pl.BlockSpec((BN, 256), blk),
                   pl.BlockSpec((64, 128), cst), pl.BlockSpec((1, 128), cst)],
        out_shape=[jax.ShapeDtypeStruct((N, 128), F32),
                   jax.ShapeDtypeStruct((N, 256), F32),
                   jax.ShapeDtypeStruct((64, 128), F32),
                   jax.ShapeDtypeStruct((1, 128), F32)],
    )(x, batchf, w1s, w1d, wn1ax, we1b, wn1ae, be1b, bn1a)


def _k_b(gs, gc, ea, we1ae, be1a, m1, c1):
    def body(gs_r, gc_r, ea_r, we_r, be1a_r, m1_r, c1_r,
             t1_r, tt_r, brof_r, g1_r, a_r):
        i = pl.program_id(0)
        gs = gs_r[...]
        gc = gc_r[...]
        pre = (gs[:, 0:64] + gc[:, 0:64] + _dot(ea_r[...], we_r[...])
               + be1a_r[...])
        t1 = jnp.maximum(pre, 0.0)
        tt = jnp.maximum(gc[:, 64:192] + _dot(t1, m1_r[...]) + c1_r[...], 0.0)
        t1_r[...] = t1
        tt_r[...] = tt
        brof = gs[:, 64:65]
        brof_r[...] = brof
        iota = lax.broadcasted_iota(jnp.int32, (1, G), 1).astype(F32)
        oh = (brof == iota).astype(F32)
        cat = jnp.concatenate([t1, jnp.ones((BE, 64), F32)], axis=1)
        g1u = lax.dot_general(oh, cat, (((0,), (0,)), ((), ())),
                              preferred_element_type=F32,
                              precision=lax.Precision.HIGHEST)
        au = lax.dot_general(t1, t1, (((0,), (0,)), ((), ())),
                             preferred_element_type=F32,
                             precision=lax.Precision.HIGHEST)

        @pl.when(i == 0)
        def _():
            g1_r[...] = jnp.zeros((G, 128), F32)
            a_r[...] = jnp.zeros((64, 64), F32)

        g1_r[...] += g1u
        a_r[...] += au

    cst = lambda i: (0, 0)
    blk = lambda i: (i, 0)
    return pl.pallas_call(
        body,
        grid=(E // BE,),
        in_specs=[pl.BlockSpec((BE, 128), blk), pl.BlockSpec((BE, 256), blk),
                  pl.BlockSpec((BE, 16), blk),
                  pl.BlockSpec((16, 64), cst), pl.BlockSpec((1, 64), cst),
                  pl.BlockSpec((64, 128), cst), pl.BlockSpec((1, 128), cst)],
        out_specs=[pl.BlockSpec((BE, 64), blk), pl.BlockSpec((BE, 128), blk),
                   pl.BlockSpec((BE, 1), blk), pl.BlockSpec((G, 128), cst),
                   pl.BlockSpec((64, 64), cst)],
        out_shape=[jax.ShapeDtypeStruct((E, 64), F32),
                   jax.ShapeDtypeStruct((E, 128), F32),
                   jax.ShapeDtypeStruct((E, 1), F32),
                   jax.ShapeDtypeStruct((G, 128), F32),
                   jax.ShapeDtypeStruct((64, 64), F32)],
    )(gs, gc, ea, we1ae, be1a, m1, c1)


def _k_d(a0, c0, x, wn1b, bn1b, wn1cx, wn1ca, bn1c, wn1d, bn1d):
    def body(a0_r, c0_r, x_r, wn1b_r, bn1b_r, wcx_r, wca_r,
             bn1c_r, wn1d_r, bn1d_r, x2_r, xs_r):
        i = pl.program_id(0)
        cnt = c0_r[...][:, 0:1]
        cc = jnp.maximum(cnt, 1.0)
        maskn = (cnt > 0).astype(F32)
        mt = a0_r[...] / cc
        agg = _dot(mt, wn1b_r[...]) + maskn * bn1b_r[...]
        h1 = jnp.maximum(
            _dot(x_r[...], wcx_r[...]) + _dot(agg, wca_r[...]) + bn1c_r[...], 0.0)
        x2 = _dot(h1, wn1d_r[...]) + bn1d_r[...]
        x2_r[...] = x2

        @pl.when(i == 0)
        def _():
            xs_r[...] = jnp.zeros((2, 256), F32)

        xs_r[...] += jnp.concatenate(
            [jnp.sum(x2, axis=0, keepdims=True),
             jnp.sum(x2 * x2, axis=0, keepdims=True)], axis=0)

    cst = lambda i: (0, 0)
    blk = lambda i: (i, 0)
    return pl.pallas_call(
        body,
        grid=(N // BN,),
        in_specs=[pl.BlockSpec((BN, 128), blk), pl.BlockSpec((BN, 128), blk),
                  pl.BlockSpec((BN, 128), blk),
                  pl.BlockSpec((128, 128), cst), pl.BlockSpec((1, 128), cst),
                  pl.BlockSpec((128, 128), cst), pl.BlockSpec((128, 128), cst),
                  pl.BlockSpec((1, 128), cst), pl.BlockSpec((128, 256), cst),
                  pl.BlockSpec((1, 256), cst)],
        out_specs=[pl.BlockSpec((BN, 256), blk), pl.BlockSpec((2, 256), cst)],
        out_shape=[jax.ShapeDtypeStruct((N, 256), F32),
                   jax.ShapeDtypeStruct((2, 256), F32)],
    )(a0, c0, x, wn1b, bn1b, wn1cx, wn1ca, bn1c, wn1d, bn1d)


def _k_d2(x2, xs, we2as, we2ad, gn, bnn):
    def body(x2_r, xs_r, ws_r, wd_r, gn_r, bnn_r, qs_r, qd_r, csd_r):
        xs = xs_r[...]
        mx = xs[0:1] * (1.0 / N)
        vx = xs[1:2] * (1.0 / N) - mx * mx
        sx = gn_r[...] * lax.rsqrt(vx + 1e-5)
        tn = bnn_r[...] - mx * sx
        x2s = x2_r[...] * sx
        qs_r[...] = _dot(x2s, ws_r[...])
        qd_r[...] = _dot(x2s, wd_r[...])
        csd_r[...] = _dot(tn, ws_r[...]) + _dot(tn, wd_r[...])

    cst = lambda i: (0, 0)
    blk = lambda i: (i, 0)
    return pl.pallas_call(
        body,
        grid=(N // BN,),
        in_specs=[pl.BlockSpec((BN, 256), blk), pl.BlockSpec((2, 256), cst),
                  pl.BlockSpec((256, 128), cst), pl.BlockSpec((256, 128), cst),
                  pl.BlockSpec((1, 256), cst), pl.BlockSpec((1, 256), cst)],
        out_specs=[pl.BlockSpec((BN, 128), blk), pl.BlockSpec((BN, 128), blk),
                   pl.BlockSpec((1, 128), cst)],
        out_shape=[jax.ShapeDtypeStruct((N, 128), F32),
                   jax.ShapeDtypeStruct((N, 128), F32),
                   jax.ShapeDtypeStruct((1, 128), F32)],
    )(x2, xs, we2as, we2ad, gn, bnn)


def _k_w(g1acc, aacc, we1b, be1b, ge, bee, we2ae, be2a):
    def body(g1_r, a_r, we1b_r, be1b_r, ge_r, bee_r, wee_r, be2a_r, m2_r, c2e_r):
        we1b = we1b_r[...]
        be1b = be1b_r[...]
        s1 = jnp.sum(g1_r[...][:, 0:64], axis=0, keepdims=True)
        s1w = _dot(s1, we1b)
        me = (s1w + E * be1b) * (1.0 / E)
        d1 = _dot(a_r[...], we1b)
        diag = jnp.sum(we1b * d1, axis=0, keepdims=True)
        sumsq = diag + 2.0 * be1b * s1w + E * be1b * be1b
        ve = sumsq * (1.0 / E) - me * me
        se = ge_r[...] * lax.rsqrt(ve + 1e-5)
        te = bee_r[...] - me * se
        m2_r[...] = _dot(we1b * se, wee_r[...])
        c2e_r[...] = _dot(be1b * se + te, wee_r[...]) + be2a_r[...]

    return pl.pallas_call(
        body,
        out_shape=[jax.ShapeDtypeStruct((64, 128), F32),
                   jax.ShapeDtypeStruct((1, 128), F32)],
    )(g1acc, aacc, we1b, be1b, ge, bee, we2ae, be2a)


def _k_e(g2s, g2d, t1, brof, m2, c2e, c2sd):
    def body(gs_r, gd_r, t1_r, brof_r, m2_r, c2e_r, csd_r, g2_r):
        i = pl.program_id(0)
        pre = (gs_r[...] + gd_r[...] + _dot(t1_r[...], m2_r[...])
               + c2e_r[...] + csd_r[...])
        t2 = jnp.maximum(pre, 0.0)
        iota = lax.broadcasted_iota(jnp.int32, (1, G), 1).astype(F32)
        oh = (brof_r[...] == iota).astype(F32)
        g2u = lax.dot_general(oh, t2, (((0,), (0,)), ((), ())),
                              preferred_element_type=F32,
                              precision=lax.Precision.HIGHEST)

        @pl.when(i == 0)
        def _():
            g2_r[...] = jnp.zeros((G, 128), F32)

        g2_r[...] += g2u

    cst = lambda i: (0, 0)
    blk = lambda i: (i, 0)
    return pl.pallas_call(
        body,
        grid=(E // BE,),
        in_specs=[pl.BlockSpec((BE, 128), blk), pl.BlockSpec((BE, 128), blk),
                  pl.BlockSpec((BE, 64), blk), pl.BlockSpec((BE, 1), blk),
                  pl.BlockSpec((64, 128), cst), pl.BlockSpec((1, 128), cst),
                  pl.BlockSpec((1, 128), cst)],
        out_specs=[pl.BlockSpec((G, 128), cst)],
        out_shape=[jax.ShapeDtypeStruct((G, 128), F32)],
    )(g2s, g2d, t1, brof, m2, c2e, c2sd)


def _k_f(g1acc, g2acc, we1b, be1b, we2b, be2b, wg1ag, bg1a, wg1b, bg1b,
         gu, bu, wg2au, wg2ag, bg2a, wg2b, bg2b, wf1, bf1, wf2, bf2):
    def body(g1_r, g2_r, we1b_r, be1b_r, we2b_r, be2b_r, wg1ag_r, bg1a_r,
             wg1b_r, bg1b_r, gu_r, bu_r, wg2au_r, wg2ag_r, bg2a_r, wg2b_r,
             bg2b_r, wf1_r, bf1_r, wf2_r, bf2_r, out_r):
        g1acc = g1_r[...]
        gcnt = g1acc[:, 64:65]
        gc = jnp.maximum(gcnt, 1.0)
        mask = (gcnt > 0).astype(F32)
        g1 = _dot(g1acc[:, 0:64] / gc, we1b_r[...]) + mask * be1b_r[...]
        u1h = jnp.maximum(_dot(g1, wg1ag_r[...]) + bg1a_r[...], 0.0)
        u1 = _dot(u1h, wg1b_r[...]) + bg1b_r[...]
        m = jnp.mean(u1, axis=0, keepdims=True)
        v = jnp.mean((u1 - m) * (u1 - m), axis=0, keepdims=True)
        u1 = (u1 - m) * lax.rsqrt(v + 1e-5) * gu_r[...] + bu_r[...]
        g2 = _dot(g2_r[...] / gc, we2b_r[...]) + mask * be2b_r[...]
        u2h = jnp.maximum(
            _dot(u1, wg2au_r[...]) + _dot(g2, wg2ag_r[...]) + bg2a_r[...], 0.0)
        u2 = _dot(u2h, wg2b_r[...]) + bg2b_r[...]
        f1 = jnp.maximum(_dot(u2, wf1_r[...]) + bf1_r[...], 0.0)
        out_r[...] = _dot(f1, wf2_r[...]) + bf2_r[...]

    return pl.pallas_call(
        body,
        out_shape=jax.ShapeDtypeStruct((G, 1), F32),
    )(g1acc, g2acc, we1b, be1b, we2b, be2b, wg1ag, bg1a, wg1b, bg1b,
      gu, bu, wg2au, wg2ag, bg2a, wg2b, bg2b, wf1, bf1, wf2, bf2)


def kernel(x, edge_attr, params, edge_index, batch, num_graphs):
    p = params
    assert x.shape == (N, 128) and edge_index.shape == (2, E)

    row = edge_index[0]
    col = edge_index[1]
    batchf = batch.astype(F32).reshape(N, 1)

    r1 = lambda b: b.reshape(1, -1)

    r128, tc, m1, c1 = _k_a(
        x, batchf,
        p["We1a"][0:128], p["We1a"][128:256], p["Wn1a"][0:128],
        p["We1b"], p["Wn1a"][128:256], r1(p["be1b"]), r1(p["bn1a"]))

    gs, gc = _sc_gather_l1(row, col, r128, tc)

    t1, tt, brof, g1acc, aacc = _k_b(
        gs, gc, edge_attr,
        p["We1a"][256:272], r1(p["be1a"]), m1, c1)

    z128 = jnp.zeros((CH, 128), F32)
    o128 = jnp.ones((CH, 128), F32)
    a0, cn0 = _sc_scatter(row, tt, z128, o128)

    x2, xs = _k_d(
        a0, cn0, x,
        p["Wn1b"], r1(p["bn1b"]), p["Wn1c"][0:128], p["Wn1c"][128:256],
        r1(p["bn1c"]), p["Wn1d"], r1(p["bn1d"]))

    q2s, q2d, c2sd = _k_d2(
        x2, xs, p["We2a"][0:256], p["We2a"][256:512],
        r1(p["g_node"]), r1(p["b_node"]))

    m2, c2e = _k_w(
        g1acc, aacc, p["We1b"], r1(p["be1b"]),
        r1(p["g_edge"]), r1(p["b_edge"]), p["We2a"][512:640], r1(p["be2a"]))

    g2s, g2d = _sc_gather_l2(row, col, q2s, q2d)

    (g2acc,) = _k_e(g2s, g2d, t1, brof, m2, c2e, c2sd)

    out = _k_f(
        g1acc, g2acc, p["We1b"], r1(p["be1b"]), p["We2b"], r1(p["be2b"]),
        p["Wg1a"][64:192], r1(p["bg1a"]), p["Wg1b"], r1(p["bg1b"]),
        r1(p["g_u"]), r1(p["b_u"]),
        p["Wg2a"][0:64], p["Wg2a"][64:192], r1(p["bg2a"]), p["Wg2b"],
        r1(p["bg2b"]), p["Wf1"], r1(p["bf1"]), p["Wf2"], r1(p["bf2"]))
    return out


# speedup vs baseline: 3.4374x; 1.0855x over previous
import functools

import jax
import jax.numpy as jnp
from jax import lax
from jax.experimental import pallas as pl
from jax.experimental.pallas import tpu as pltpu
from jax.experimental.pallas import tpu_sc as plsc

F32 = jnp.float32

N = 10000
E = 320000
G = 64

BN = 1000
BE = 4000

NC, NS = 2, 16
W = NC * NS
EW = E // W
C = 400
NCH = EW // C
RT = N // NS


def _mesh():
    return plsc.VectorSubcoreMesh(
        core_axis_name="c", subcore_axis_name="s", num_cores=NC, num_subcores=NS)


CH = 128
NCHT = E // CH


def _sc_gather2(row, col, ta, tb, wa, wb):
    def body(row_h, col_h, a_h, b_h, oa_o, ob_o,
             idr0, idr1, idc0, idc1, ba0, ba1, bb0, bb1, sg0, sg1, sw0, sw1):
        idr, idc = (idr0, idr1), (idc0, idc1)
        ba, bb = (ba0, ba1), (bb0, bb1)
        sg, sw = (sg0, sg1), (sw0, sw1)
        wid = lax.axis_index("s") * NC + lax.axis_index("c")
        nch = NCHT // W + jnp.where(wid < NCHT % W, 1, 0)

        def base_of(k):
            return (k * W + wid) * CH

        def load_start(k, b):
            base = base_of(k)
            pltpu.sync_copy(row_h.at[pl.ds(base, CH)], idr[b])
            pltpu.sync_copy(col_h.at[pl.ds(base, CH)], idc[b])
            pltpu.async_copy(a_h.at[idr[b]], ba[b], sg[b])
            pltpu.async_copy(b_h.at[idc[b]], bb[b], sg[b])

        def gather_wait(b):
            pltpu.make_async_copy(a_h.at[idr[b]], ba[b], sg[b]).wait()
            pltpu.make_async_copy(b_h.at[idc[b]], bb[b], sg[b]).wait()

        def write_start(k, b):
            base = base_of(k)
            pltpu.async_copy(ba[b], oa_o.at[pl.ds(base, CH)], sw[b])
            pltpu.async_copy(bb[b], ob_o.at[pl.ds(base, CH)], sw[b])

        def write_wait(k, b):
            base = base_of(k)
            pltpu.make_async_copy(ba[b], oa_o.at[pl.ds(base, CH)], sw[b]).wait()
            pltpu.make_async_copy(bb[b], ob_o.at[pl.ds(base, CH)], sw[b]).wait()

        for b in range(2):
            @pl.when(b < nch)
            def _(b=b):
                load_start(b, b)

        def outer(ss, carry):
            for b in range(2):
                k = 2 * ss + b

                @pl.when(k < nch)
                def _(b=b, k=k):
                    gather_wait(b)
                    write_start(k, b)
                    write_wait(k, b)

                    @pl.when(k + 2 < nch)
                    def _(b=b, k=k):
                        load_start(k + 2, b)
            return carry

        lax.fori_loop(0, (nch + 1) // 2, outer, 0)

    f = pl.kernel(
        body,
        out_type=(jax.ShapeDtypeStruct((E, wa), F32),
                  jax.ShapeDtypeStruct((E, wb), F32)),
        mesh=_mesh(),
        scratch_types=[pltpu.VMEM((CH,), jnp.int32), pltpu.VMEM((CH,), jnp.int32),
                       pltpu.VMEM((CH,), jnp.int32), pltpu.VMEM((CH,), jnp.int32),
                       pltpu.VMEM((CH, wa), F32), pltpu.VMEM((CH, wa), F32),
                       pltpu.VMEM((CH, wb), F32), pltpu.VMEM((CH, wb), F32),
                       pltpu.SemaphoreType.DMA, pltpu.SemaphoreType.DMA,
                       pltpu.SemaphoreType.DMA, pltpu.SemaphoreType.DMA])
    return f(row, col, ta, tb)


def _sc_gather_l1(row, col, r128, tc):
    return _sc_gather2(row, col, r128, tc, 128, 256)


def _sc_gather_l2(row, col, q2s, q2d):
    return _sc_gather2(row, col, q2s, q2d, 128, 128)


def _remap_idx(idx, idx2, lo, nh):
    def remap(i, carry):
        v = idx[pl.ds(i * 16, 16)] - lo
        oob = (v < 0) | (v >= nh)
        trash = nh + (v & 7)
        idx2[pl.ds(i * 16, 16)] = jnp.where(oob, trash, v)
        return carry

    lax.fori_loop(0, CH // 16, remap, 0)


def _sc_scatter(row, tt, z128, o128):
    NH = N // NC
    RB = 312

    def body(row_h, tt_h, z128_h, o128_h, a_o, c_o,
             idx0, idx1, idp0, idp1, buf0, buf1, obuf, accS, sl0, sl1):
        idx, idp = (idx0, idx1), (idp0, idp1)
        buf, sl = (buf0, buf1), (sl0, sl1)
        cid = lax.axis_index("c")
        sid = lax.axis_index("s")
        lo = cid * NH
        rbase = sid * RB
        nch = NCHT // NS + jnp.where(sid < NCHT % NS, 1, 0)

        def base_of(k):
            return (k * NS + sid) * CH

        def zero_window():
            pltpu.sync_copy(z128_h, buf0)
            for (o, sz) in ((0, 128), (128, 128), (256, 72)):
                pltpu.sync_copy(buf0.at[pl.ds(0, sz)], accS.at[pl.ds(rbase + o, sz)])

        def stage_out(dst):
            for (o, sz) in ((0, 128), (128, 128), (256, 64)):
                pltpu.sync_copy(accS.at[pl.ds(rbase + o, sz)], buf0.at[pl.ds(0, sz)])
                pltpu.sync_copy(buf0.at[pl.ds(0, sz)], dst.at[pl.ds(lo + rbase + o, sz)])

        pltpu.sync_copy(o128_h, obuf)
        zero_window()
        plsc.subcore_barrier()

        def load1(k, b):
            base = base_of(k)
            pltpu.async_copy(row_h.at[pl.ds(base, CH)], idx[b], sl[b])
            pltpu.async_copy(tt_h.at[pl.ds(base, CH)], buf[b], sl[b])

        def wait1(k, b):
            base = base_of(k)
            pltpu.make_async_copy(row_h.at[pl.ds(base, CH)], idx[b], sl[b]).wait()
            pltpu.make_async_copy(tt_h.at[pl.ds(base, CH)], buf[b], sl[b]).wait()

        for b in range(2):
            @pl.when(b < nch)
            def _(b=b):
                load1(b, b)

        def outer1(ss, carry):
            for b in range(2):
                k = 2 * ss + b

                @pl.when(k < nch)
                def _(b=b, k=k):
                    wait1(k, b)
                    _remap_idx(idx[b], idp[b], lo, NH)
                    pltpu.sync_copy(buf[b], accS.at[idp[b]], add=True)

                    @pl.when(k + 2 < nch)
                    def _(b=b, k=k):
                        load1(k + 2, b)
            return carry

        lax.fori_loop(0, (nch + 1) // 2, outer1, 0)
        plsc.subcore_barrier()
        stage_out(a_o)
        plsc.subcore_barrier()

        zero_window()
        plsc.subcore_barrier()

        def load2(k, b):
            pltpu.async_copy(row_h.at[pl.ds(base_of(k), CH)], idx[b], sl[b])

        def wait2(k, b):
            pltpu.make_async_copy(
                row_h.at[pl.ds(base_of(k), CH)], idx[b], sl[b]).wait()

        for b in range(2):
            @pl.when(b < nch)
            def _(b=b):
                load2(b, b)

        def outer2(ss, carry):
            for b in range(2):
                k = 2 * ss + b

                @pl.when(k < nch)
                def _(b=b, k=k):
                    wait2(k, b)
                    _remap_idx(idx[b], idp[b], lo, NH)
                    pltpu.sync_copy(obuf, accS.at[idp[b]], add=True)

                    @pl.when(k + 2 < nch)
                    def _(b=b, k=k):
                        load2(k + 2, b)
            return carry

        lax.fori_loop(0, (nch + 1) // 2, outer2, 0)
        plsc.subcore_barrier()
        stage_out(c_o)

    f = pl.kernel(
        body,
        out_type=(jax.ShapeDtypeStruct((N, 128), F32),
                  jax.ShapeDtypeStruct((N, 128), F32)),
        mesh=_mesh(),
        scratch_types=[pltpu.VMEM((CH,), jnp.int32), pltpu.VMEM((CH,), jnp.int32),
                       pltpu.VMEM((CH,), jnp.int32), pltpu.VMEM((CH,), jnp.int32),
                       pltpu.VMEM((CH, 128), F32), pltpu.VMEM((CH, 128), F32),
                       pltpu.VMEM((CH, 128), F32),
                       pltpu.VMEM_SHARED((N // NC + 8, 128), F32),
                       pltpu.SemaphoreType.DMA, pltpu.SemaphoreType.DMA])
    return f(row, tt, z128, o128)


def _dot(a, b):
    return jnp.dot(a, b, preferred_element_type=F32,
                   precision=lax.Precision.HIGHEST)


def _dotd(a, b):
    return jnp.dot(a, b, preferred_element_type=F32)


def _k_a(x, batchf, w1s, w1d, wn1ax, we1b, wn1ae, be1b, bn1a):
    def body(x_r, bf_r, w1s_r, w1d_r, wnx_r, we1b_r, wne_r, be1b_r, bn1a_r,
             r128_r, tc_r, m1_r, c1_r):
        xb = x_r[...]
        r128_r[...] = jnp.concatenate(
            [_dotd(xb, w1s_r[...]), bf_r[...], jnp.zeros((BN, 63), F32)], axis=1)
        tc_r[...] = jnp.concatenate(
            [_dotd(xb, w1d_r[...]), _dot(xb, wnx_r[...]),
             jnp.zeros((BN, 64), F32)], axis=1)
        m1_r[...] = _dot(we1b_r[...], wne_r[...])
        c1_r[...] = _dot(be1b_r[...], wne_r[...]) + bn1a_r[...]

    cst = lambda i: (0, 0)
    blk = lambda i: (i, 0)
    return pl.pallas_call(
        body,
        grid=(N // BN,),
        in_specs=[pl.BlockSpec((BN, 128), blk), pl.BlockSpec((BN, 1), blk),
                  pl.BlockSpec((128, 64), cst), pl.BlockSpec((128, 64), cst),
                  pl.BlockSpec((128, 128), cst), pl.BlockSpec((64, 128), cst),
                  pl.BlockSpec((128, 128), cst), pl.BlockSpec((1, 128), cst),
                  pl.BlockSpec((1, 128), cst)],
        out_specs=[pl.BlockSpec((BN, 128), blk), pl.BlockSpec((BN, 256), blk),
                   pl.BlockSpec((64, 128), cst), pl.BlockSpec((1, 128), cst)],
        out_shape=[jax.ShapeDtypeStruct((N, 128), F32),
                   jax.ShapeDtypeStruct((N, 256), F32),
                   jax.ShapeDtypeStruct((64, 128), F32),
                   jax.ShapeDtypeStruct((1, 128), F32)],
    )(x, batchf, w1s, w1d, wn1ax, we1b, wn1ae, be1b, bn1a)


def _k_b(gs, gc, ea, we1ae, be1a, we1b, be1b, m1, c1):
    def body(gs_r, gc_r, ea_r, we_r, be1a_r, we1b_r, be1b_r, m1_r, c1_r,
             t1_r, tt_r, brof_r, g1_r, a_r):
        i = pl.program_id(0)
        gs = gs_r[...]
        gc = gc_r[...]
        pre = (gs[:, 0:64] + gc[:, 0:64] + _dotd(ea_r[...], we_r[...])
               + be1a_r[...])
        t1 = jnp.maximum(pre, 0.0)
        tt = jnp.maximum(gc[:, 64:192] + _dot(t1, m1_r[...]) + c1_r[...], 0.0)
        ea2 = _dotd(t1, we1b_r[...]) + be1b_r[...]
        t1_r[...] = t1
        tt_r[...] = tt
        brof = gs[:, 64:65]
        brof_r[...] = brof
        iota = lax.broadcasted_iota(jnp.int32, (1, G), 1).astype(F32)
        oh = (brof == iota).astype(F32)
        cat = jnp.concatenate([ea2, jnp.ones((BE, 128), F32)], axis=1)
        g1u = lax.dot_general(oh, cat, (((0,), (0,)), ((), ())),
                              preferred_element_type=F32,
                              precision=lax.Precision.HIGHEST)
        au = lax.dot_general(t1, t1, (((0,), (0,)), ((), ())),
                             preferred_element_type=F32,
                             precision=lax.Precision.HIGHEST)

        @pl.when(i == 0)
        def _():
            g1_r[...] = jnp.zeros((G, 256), F32)
            a_r[...] = jnp.zeros((64, 64), F32)

        g1_r[...] += g1u
        a_r[...] += au

    cst = lambda i: (0, 0)
    blk = lambda i: (i, 0)
    return pl.pallas_call(
        body,
        grid=(E // BE,),
        in_specs=[pl.BlockSpec((BE, 128), blk), pl.BlockSpec((BE, 256), blk),
                  pl.BlockSpec((BE, 16), blk),
                  pl.BlockSpec((16, 64), cst), pl.BlockSpec((1, 64), cst),
                  pl.BlockSpec((64, 128), cst), pl.BlockSpec((1, 128), cst),
                  pl.BlockSpec((64, 128), cst), pl.BlockSpec((1, 128), cst)],
        out_specs=[pl.BlockSpec((BE, 64), blk), pl.BlockSpec((BE, 128), blk),
                   pl.BlockSpec((BE, 1), blk), pl.BlockSpec((G, 256), cst),
                   pl.BlockSpec((64, 64), cst)],
        out_shape=[jax.ShapeDtypeStruct((E, 64), F32),
                   jax.ShapeDtypeStruct((E, 128), F32),
                   jax.ShapeDtypeStruct((E, 1), F32),
                   jax.ShapeDtypeStruct((G, 256), F32),
                   jax.ShapeDtypeStruct((64, 64), F32)],
    )(gs, gc, ea, we1ae, be1a, we1b, be1b, m1, c1)


def _k_d(a0, c0, x, wn1b, bn1b, wn1cx, wn1ca, bn1c, wn1d, bn1d):
    def body(a0_r, c0_r, x_r, wn1b_r, bn1b_r, wcx_r, wca_r,
             bn1c_r, wn1d_r, bn1d_r, x2_r, xs_r):
        i = pl.program_id(0)
        cnt = c0_r[...][:, 0:1]
        cc = jnp.maximum(cnt, 1.0)
        maskn = (cnt > 0).astype(F32)
        mt = a0_r[...] / cc
        agg = _dot(mt, wn1b_r[...]) + maskn * bn1b_r[...]
        h1 = jnp.maximum(
            _dot(x_r[...], wcx_r[...]) + _dot(agg, wca_r[...]) + bn1c_r[...], 0.0)
        x2 = _dot(h1, wn1d_r[...]) + bn1d_r[...]
        x2_r[...] = x2

        @pl.when(i == 0)
        def _():
            xs_r[...] = jnp.zeros((2, 256), F32)

        xs_r[...] += jnp.concatenate(
            [jnp.sum(x2, axis=0, keepdims=True),
             jnp.sum(x2 * x2, axis=0, keepdims=True)], axis=0)

    cst = lambda i: (0, 0)
    blk = lambda i: (i, 0)
    return pl.pallas_call(
        body,
        grid=(N // BN,),
        in_specs=[pl.BlockSpec((BN, 128), blk), pl.BlockSpec((BN, 128), blk),
                  pl.BlockSpec((BN, 128), blk),
                  pl.BlockSpec((128, 128), cst), pl.BlockSpec((1, 128), cst),
                  pl.BlockSpec((128, 128), cst), pl.BlockSpec((128, 128), cst),
                  pl.BlockSpec((1, 128), cst), pl.BlockSpec((128, 256), cst),
                  pl.BlockSpec((1, 256), cst)],
        out_specs=[pl.BlockSpec((BN, 256), blk), pl.BlockSpec((2, 256), cst)],
        out_shape=[jax.ShapeDtypeStruct((N, 256), F32),
                   jax.ShapeDtypeStruct((2, 256), F32)],
    )(a0, c0, x, wn1b, bn1b, wn1cx, wn1ca, bn1c, wn1d, bn1d)


def _k_d2(x2, xs, we2as, we2ad, gn, bnn):
    def body(x2_r, xs_r, ws_r, wd_r, gn_r, bnn_r, qs_r, qd_r, csd_r):
        xs = xs_r[...]
        mx = xs[0:1] * (1.0 / N)
        vx = xs[1:2] * (1.0 / N) - mx * mx
        sx = gn_r[...] * lax.rsqrt(vx + 1e-5)
        tn = bnn_r[...] - mx * sx
        x2s = x2_r[...] * sx
        qs_r[...] = _dot(x2s, ws_r[...])
        qd_r[...] = _dot(x2s, wd_r[...])
        csd_r[...] = _dot(tn, ws_r[...]) + _dot(tn, wd_r[...])

    cst = lambda i: (0, 0)
    blk = lambda i: (i, 0)
    return pl.pallas_call(
        body,
        grid=(N // BN,),
        in_specs=[pl.BlockSpec((BN, 256), blk), pl.BlockSpec((2, 256), cst),
                  pl.BlockSpec((256, 128), cst), pl.BlockSpec((256, 128), cst),
                  pl.BlockSpec((1, 256), cst), pl.BlockSpec((1, 256), cst)],
        out_specs=[pl.BlockSpec((BN, 128), blk), pl.BlockSpec((BN, 128), blk),
                   pl.BlockSpec((1, 128), cst)],
        out_shape=[jax.ShapeDtypeStruct((N, 128), F32),
                   jax.ShapeDtypeStruct((N, 128), F32),
                   jax.ShapeDtypeStruct((1, 128), F32)],
    )(x2, xs, we2as, we2ad, gn, bnn)


def _k_w(g1acc, aacc, we1b, be1b, ge, bee, we2ae, be2a):
    def body(g1_r, a_r, we1b_r, be1b_r, ge_r, bee_r, wee_r, be2a_r, m2_r, c2e_r):
        we1b = we1b_r[...]
        be1b = be1b_r[...]
        sum_ea2 = jnp.sum(g1_r[...][:, 0:128], axis=0, keepdims=True)
        me = sum_ea2 * (1.0 / E)
        s1w = sum_ea2 - E * be1b
        d1 = _dot(a_r[...], we1b)
        diag = jnp.sum(we1b * d1, axis=0, keepdims=True)
        sumsq = diag + 2.0 * be1b * s1w + E * be1b * be1b
        ve = sumsq * (1.0 / E) - me * me
        se = ge_r[...] * lax.rsqrt(ve + 1e-5)
        te = bee_r[...] - me * se
        m2_r[...] = _dot(we1b * se, wee_r[...])
        c2e_r[...] = _dot(be1b * se + te, wee_r[...]) + be2a_r[...]

    return pl.pallas_call(
        body,
        out_shape=[jax.ShapeDtypeStruct((64, 128), F32),
                   jax.ShapeDtypeStruct((1, 128), F32)],
    )(g1acc, aacc, we1b, be1b, ge, bee, we2ae, be2a)


def _k_e(g2s, g2d, t1, brof, m2, c2e, c2sd):
    def body(gs_r, gd_r, t1_r, brof_r, m2_r, c2e_r, csd_r, g2_r):
        i = pl.program_id(0)
        pre = (gs_r[...] + gd_r[...] + _dot(t1_r[...], m2_r[...])
               + c2e_r[...] + csd_r[...])
        t2 = jnp.maximum(pre, 0.0)
        iota = lax.broadcasted_iota(jnp.int32, (1, G), 1).astype(F32)
        oh = (brof_r[...] == iota).astype(F32)
        g2u = lax.dot_general(oh, t2, (((0,), (0,)), ((), ())),
                              preferred_element_type=F32,
                              precision=lax.Precision.HIGHEST)

        @pl.when(i == 0)
        def _():
            g2_r[...] = jnp.zeros((G, 128), F32)

        g2_r[...] += g2u

    cst = lambda i: (0, 0)
    blk = lambda i: (i, 0)
    return pl.pallas_call(
        body,
        grid=(E // BE,),
        in_specs=[pl.BlockSpec((BE, 128), blk), pl.BlockSpec((BE, 128), blk),
                  pl.BlockSpec((BE, 64), blk), pl.BlockSpec((BE, 1), blk),
                  pl.BlockSpec((64, 128), cst), pl.BlockSpec((1, 128), cst),
                  pl.BlockSpec((1, 128), cst)],
        out_specs=[pl.BlockSpec((G, 128), cst)],
        out_shape=[jax.ShapeDtypeStruct((G, 128), F32)],
    )(g2s, g2d, t1, brof, m2, c2e, c2sd)


def _k_f(g1acc, g2acc, we2b, be2b, wg1a, bg1a, wg1b, bg1b,
         gu, bu, wg2a, bg2a, wg2b, bg2b, wf1, bf1, wf2, bf2):
    def body(g1_r, g2_r, we2b_r, be2b_r, wg1a_r, bg1a_r, wg1b_r, bg1b_r,
             gu_r, bu_r, wg2a_r, bg2a_r, wg2b_r, bg2b_r, wf1_r, bf1_r,
             wf2_r, bf2_r, out_r):
        g1acc = g1_r[...]
        gcnt = g1acc[:, 128:129]
        gc = jnp.maximum(gcnt, 1.0)
        mask = (gcnt > 0).astype(F32)
        g1 = g1acc[:, 0:128] / gc
        h1 = jnp.concatenate([jnp.zeros((G, 64), F32), g1], axis=1)
        u1 = jnp.maximum(jnp.dot(h1, wg1a_r[...], preferred_element_type=F32)
                         + bg1a_r[...], 0.0)
        u1 = jnp.dot(u1, wg1b_r[...], preferred_element_type=F32) + bg1b_r[...]
        m = jnp.mean(u1, axis=0, keepdims=True)
        v = jnp.mean((u1 - m) * (u1 - m), axis=0, keepdims=True)
        u1 = (u1 - m) / jnp.sqrt(v + 1e-5) * gu_r[...] + bu_r[...]
        g2 = _dot(g2_r[...] / gc, we2b_r[...]) + mask * be2b_r[...]
        h2 = jnp.concatenate([u1, g2], axis=1)
        u2 = jnp.maximum(jnp.dot(h2, wg2a_r[...], preferred_element_type=F32)
                         + bg2a_r[...], 0.0)
        u2 = jnp.dot(u2, wg2b_r[...], preferred_element_type=F32) + bg2b_r[...]
        f1 = jnp.maximum(jnp.dot(u2, wf1_r[...], preferred_element_type=F32)
                         + bf1_r[...], 0.0)
        out_r[...] = (jnp.dot(f1, wf2_r[...], preferred_element_type=F32)
                      + bf2_r[...])

    return pl.pallas_call(
        body,
        out_shape=jax.ShapeDtypeStruct((G, 1), F32),
    )(g1acc, g2acc, we2b, be2b, wg1a, bg1a, wg1b, bg1b,
      gu, bu, wg2a, bg2a, wg2b, bg2b, wf1, bf1, wf2, bf2)


def kernel(x, edge_attr, params, edge_index, batch, num_graphs):
    p = params
    assert x.shape == (N, 128) and edge_index.shape == (2, E)

    row = edge_index[0]
    col = edge_index[1]
    batchf = batch.astype(F32).reshape(N, 1)

    r1 = lambda b: b.reshape(1, -1)

    r128, tc, m1, c1 = _k_a(
        x, batchf,
        p["We1a"][0:128], p["We1a"][128:256], p["Wn1a"][0:128],
        p["We1b"], p["Wn1a"][128:256], r1(p["be1b"]), r1(p["bn1a"]))

    gs, gc = _sc_gather_l1(row, col, r128, tc)

    t1, tt, brof, g1acc, aacc = _k_b(
        gs, gc, edge_attr,
        p["We1a"][256:272], r1(p["be1a"]), p["We1b"], r1(p["be1b"]), m1, c1)

    z128 = jnp.zeros((CH, 128), F32)
    o128 = jnp.ones((CH, 128), F32)
    a0, cn0 = _sc_scatter(row, tt, z128, o128)

    x2, xs = _k_d(
        a0, cn0, x,
        p["Wn1b"], r1(p["bn1b"]), p["Wn1c"][0:128], p["Wn1c"][128:256],
        r1(p["bn1c"]), p["Wn1d"], r1(p["bn1d"]))

    q2s, q2d, c2sd = _k_d2(
        x2, xs, p["We2a"][0:256], p["We2a"][256:512],
        r1(p["g_node"]), r1(p["b_node"]))

    m2, c2e = _k_w(
        g1acc, aacc, p["We1b"], r1(p["be1b"]),
        r1(p["g_edge"]), r1(p["b_edge"]), p["We2a"][512:640], r1(p["be2a"]))

    g2s, g2d = _sc_gather_l2(row, col, q2s, q2d)

    (g2acc,) = _k_e(g2s, g2d, t1, brof, m2, c2e, c2sd)

    out = _k_f(
        g1acc, g2acc, p["We2b"], r1(p["be2b"]),
        p["Wg1a"], r1(p["bg1a"]), p["Wg1b"], r1(p["bg1b"]),
        r1(p["g_u"]), r1(p["b_u"]),
        p["Wg2a"], r1(p["bg2a"]), p["Wg2b"],
        r1(p["bg2b"]), p["Wf1"], r1(p["bf1"]), p["Wf2"], r1(p["bf2"]))
    return out
